# Initial kernel scaffold; baseline (speedup 1.0000x reference)
import jax
import jax.numpy as jnp
from jax.experimental import pallas as pl


def kernel(logits, target):
    raise NotImplementedError("write your pallas kernel here")


# speedup vs baseline: 12.8097x; 12.8097x over previous
import jax
import jax.numpy as jnp
from jax.experimental import pallas as pl
from jax.experimental.pallas import tpu as pltpu
from functools import partial

_SMOOTH = 1e-05
_SENTINEL = 0x3FFFFFFF


def _dice_kernel(logits_ref, target_ref, eps_ref, out_ref):
    l = logits_ref[0]
    d = l[1] - l[0]
    p = 1.0 / (1.0 + jnp.exp(-d))
    t = target_ref[0]
    fg = t == 1
    tf = jnp.where(fg, 1.0, 0.0)
    tp = p * (tf + eps_ref[0])
    bits = jax.lax.bitcast_convert_type(tp, jnp.int32)
    fgbits = jnp.where(fg, bits, jnp.int32(_SENTINEL))

    n_fg = jnp.sum(fg.astype(jnp.int32))
    k = jnp.maximum(1, (n_fg.astype(jnp.float32) * 0.5).astype(jnp.int32))

    def step(i, res):
        cand = res | (jnp.int32(1) << (jnp.int32(29) - i))
        cnt = jnp.sum((fgbits < cand).astype(jnp.int32))
        return jnp.where(cnt <= k - 1, cand, res)

    thr = jax.lax.fori_loop(0, 30, step, jnp.int32(0))

    kept = fg & (bits <= thr)
    ign = fg & (bits > thr)
    inter = jnp.sum(jnp.where(kept, p, 0.0))
    p2 = p * p
    ssp = jnp.sum(p2) - jnp.sum(jnp.where(ign, p2, 0.0))
    sst = jnp.sum(jnp.where(kept, 1.0, 0.0))
    dice = (2.0 * inter + _SMOOTH) / (ssp + sst + _SMOOTH)
    out_ref[0] = jnp.full((8, 128), dice, dtype=jnp.float32)


@jax.jit
def kernel(logits, target):
    B = logits.shape[0]
    lg = logits.reshape(B, 2, 2048, 128)
    tg = target.reshape(B, 2048, 128)
    eps_key = jax.random.fold_in(jax.random.key(1), 7)
    eps = (jax.random.uniform(eps_key, (B, 262144), dtype=jnp.float32) * 1e-06
           ).reshape(B, 2048, 128)
    dice = pl.pallas_call(
        _dice_kernel,
        grid=(B,),
        in_specs=[
            pl.BlockSpec((1, 2, 2048, 128), lambda i: (i, 0, 0, 0)),
            pl.BlockSpec((1, 2048, 128), lambda i: (i, 0, 0)),
            pl.BlockSpec((1, 2048, 128), lambda i: (i, 0, 0)),
        ],
        out_specs=pl.BlockSpec((1, 8, 128), lambda i: (i, 0, 0)),
        out_shape=jax.ShapeDtypeStruct((B, 8, 128), jnp.float32),
    )(lg, tg, eps)
    return 1.0 - jnp.mean(dice[:, 0, 0])
